# baseline (speedup 1.0000x reference)
import functools

import jax
import jax.numpy as jnp
from jax import lax
from jax.experimental import pallas as pl
from jax.experimental.pallas import tpu as pltpu
from jax.experimental.pallas import tpu_sc as plsc

_LANES = 128
_R = 256


def _round_up(x, m):
    return (x + m - 1) // m * m


def _num_workers():
    info = plsc.get_sparse_core_info()
    return info.num_cores, info.num_subcores


def _sc_gather(table, idx):
    e_pad = idx.shape[0]
    d = table.shape[1]
    nc, ns = _num_workers()
    nw = nc * ns
    per_w = e_pad // nw
    assert e_pad % (nw * _LANES) == 0
    n_ch = per_w // _LANES
    mesh = plsc.VectorSubcoreMesh(core_axis_name="c", subcore_axis_name="s")

    @functools.partial(
        pl.kernel,
        mesh=mesh,
        out_type=jax.ShapeDtypeStruct((e_pad, d), jnp.float32),
        scratch_types=[
            pltpu.VMEM((_LANES,), jnp.int32),
            pltpu.VMEM((_LANES, d), jnp.float32),
            pltpu.SemaphoreType.DMA,
        ],
    )
    def k(table_hbm, idx_hbm, out_hbm, idx_v, rows_v, sem):
        wid = lax.axis_index("s") * nc + lax.axis_index("c")
        base = wid * per_w

        def body(i, carry):
            start = base + i * _LANES
            pltpu.sync_copy(idx_hbm.at[pl.ds(start, _LANES)], idx_v)
            pltpu.async_copy(table_hbm.at[idx_v], rows_v, sem).wait()
            pltpu.sync_copy(rows_v, out_hbm.at[pl.ds(start, _LANES)])
            return carry

        lax.fori_loop(0, n_ch, body, 0)

    return k(table, idx)


def _sc_scatter(rows, idx3):
    n_pad, d = rows.shape
    nw, kk, _ = idx3.shape
    per_w = kk * _LANES
    nc, ns = _num_workers()
    assert nw == nc * ns and n_pad == nw * per_w
    mesh = plsc.VectorSubcoreMesh(core_axis_name="c", subcore_axis_name="s")

    @functools.partial(
        pl.kernel,
        mesh=mesh,
        out_type=jax.ShapeDtypeStruct((n_pad, d), jnp.float32),
        scratch_types=[
            pltpu.VMEM((kk, _LANES), jnp.int32),
            pltpu.VMEM((_LANES, d), jnp.float32),
            pltpu.SemaphoreType.DMA,
        ],
    )
    def k(rows_hbm, idx_hbm, out_hbm, idx_v, buf_v, sem):
        wid = lax.axis_index("s") * nc + lax.axis_index("c")
        pltpu.sync_copy(idx_hbm.at[wid], idx_v)
        for c in range(kk):
            pltpu.sync_copy(
                rows_hbm.at[pl.ds(wid * per_w + c * _LANES, _LANES)], buf_v)
            pltpu.async_copy(buf_v, out_hbm.at[idx_v.at[c]], sem).wait()

    return k(rows, idx3)


def _lstm_packed(deg_sorted, x_tm, wih_t, whh_t, bias, n_pad, interpret=False):
    e_pad, d = x_tm.shape
    h = whh_t.shape[0]

    def body(deg_ref, x_hbm, wih_ref, whh_ref, b_ref, h_ref, c_ref, xbuf, sem):
        h_ref[...] = jnp.zeros_like(h_ref)
        c_ref[...] = jnp.zeros_like(c_ref)
        max_deg = deg_ref[0]

        def t_cond(s):
            return s[0] < max_deg

        def t_body(s):
            t, ptr, cnt = s

            def c_cond(c):
                return jnp.logical_and(c > 0, deg_ref[c - 1] <= t)

            cnt = lax.while_loop(c_cond, lambda c: c - 1, cnt)
            nch = (cnt + (_R - 1)) // _R

            def chunk(ci, carry):
                row0 = ci * _R
                cp = pltpu.make_async_copy(
                    x_hbm.at[pl.ds(ptr + row0, _R)], xbuf, sem)
                cp.start()
                cp.wait()
                x = xbuf[...]
                hs = h_ref[pl.ds(row0, _R), :]
                cs = c_ref[pl.ds(row0, _R), :]
                g = jnp.dot(x, wih_ref[...], preferred_element_type=jnp.float32)
                g = g + jnp.dot(hs, whh_ref[...],
                                preferred_element_type=jnp.float32)
                g = g + b_ref[...]
                gi = jax.nn.sigmoid(g[:, :h])
                gf = jax.nn.sigmoid(g[:, h:2 * h])
                gg = jnp.tanh(g[:, 2 * h:3 * h])
                go = jax.nn.sigmoid(g[:, 3 * h:])
                c_new = gf * cs + gi * gg
                h_new = go * jnp.tanh(c_new)
                m = (row0 + lax.broadcasted_iota(jnp.int32, (_R, 1), 0)) < cnt
                h_ref[pl.ds(row0, _R), :] = jnp.where(m, h_new, hs)
                c_ref[pl.ds(row0, _R), :] = jnp.where(m, c_new, cs)
                return carry

            lax.fori_loop(0, nch, chunk, 0)
            return (t + 1, ptr + cnt, cnt)

        lax.while_loop(
            t_cond, t_body,
            (jnp.int32(0), jnp.int32(0), jnp.int32(n_pad)))

    return pl.pallas_call(
        body,
        in_specs=[
            pl.BlockSpec(memory_space=pltpu.SMEM),
            pl.BlockSpec(memory_space=pl.ANY),
            pl.BlockSpec(memory_space=pltpu.VMEM),
            pl.BlockSpec(memory_space=pltpu.VMEM),
            pl.BlockSpec(memory_space=pltpu.VMEM),
        ],
        out_specs=pl.BlockSpec(memory_space=pltpu.VMEM),
        out_shape=jax.ShapeDtypeStruct((n_pad, h), jnp.float32),
        scratch_shapes=[
            pltpu.VMEM((n_pad, h), jnp.float32),
            pltpu.VMEM((_R, d), jnp.float32),
            pltpu.SemaphoreType.DMA,
        ],
        interpret=interpret,
    )(deg_sorted, x_tm, wih_t, whh_t, bias)


def _out_matmul(x_pad, agg_pad, w, interpret=False):
    n_pad, d = x_pad.shape
    h = agg_pad.shape[1]
    out_f = w.shape[1]
    blk = 256

    def body(x_ref, a_ref, w_ref, o_ref):
        o_ref[...] = (
            jnp.dot(x_ref[...], w_ref[:d, :], preferred_element_type=jnp.float32)
            + jnp.dot(a_ref[...], w_ref[d:, :],
                      preferred_element_type=jnp.float32))

    return pl.pallas_call(
        body,
        grid=(n_pad // blk,),
        in_specs=[
            pl.BlockSpec((blk, d), lambda i: (i, 0)),
            pl.BlockSpec((blk, h), lambda i: (i, 0)),
            pl.BlockSpec((d + h, out_f), lambda i: (0, 0)),
        ],
        out_specs=pl.BlockSpec((blk, out_f), lambda i: (i, 0)),
        out_shape=jax.ShapeDtypeStruct((n_pad, out_f), jnp.float32),
        interpret=interpret,
    )(x_pad, agg_pad, w)


def kernel(input_matrix, W, Wih, Whh, bih, bhh, edge_index):
    n, d = input_matrix.shape
    h = Whh.shape[1]
    e = edge_index.shape[1]
    src = edge_index[0]
    dst = edge_index[1]

    nc, ns = _num_workers()
    nw = nc * ns
    lane_blk = nw * _LANES
    n_pad = _round_up(max(n, _R), lane_blk)
    e_pad = _round_up(e + _R, lane_blk)

    src_s, dst_s = lax.sort((src, dst), num_keys=1, is_stable=True)
    ar = jnp.arange(e, dtype=jnp.int32)
    brk = src_s[1:] != src_s[:-1]
    is_start = jnp.concatenate([jnp.ones((1,), bool), brk])
    is_last = jnp.concatenate([brk, jnp.ones((1,), bool)])
    seg_start = lax.cummax(jnp.where(is_start, ar, 0))
    seg_last = jnp.flip(lax.cummin(jnp.flip(jnp.where(is_last, ar, e - 1))))
    t_j = ar - seg_start
    negdeg_e = seg_start - seg_last - 1
    _, _, _, dst_packed = lax.sort(
        (t_j, negdeg_e, src_s, dst_s), num_keys=3, is_stable=False)
    gather_idx = jnp.concatenate(
        [dst_packed, jnp.zeros((e_pad - e,), jnp.int32)])

    deg = jnp.bincount(src, length=n).astype(jnp.int32)
    negdeg_n, node_order = lax.sort(
        (-deg, jnp.arange(n, dtype=jnp.int32)), num_keys=1, is_stable=True)
    deg_pad = jnp.zeros((n_pad,), jnp.int32).at[:n].set(-negdeg_n)

    x_tm = _sc_gather(input_matrix, gather_idx)

    wih_t = Wih.T
    whh_t = Whh.T
    bias = (bih + bhh).reshape(1, -1)
    h_fin = _lstm_packed(deg_pad, x_tm, wih_t, whh_t, bias, n_pad)

    scat = jnp.concatenate(
        [node_order, jnp.arange(n, n_pad, dtype=jnp.int32)]
    ).reshape(nw, -1, _LANES)
    agg = _sc_scatter(h_fin, scat)

    x_pad = jnp.zeros((n_pad, d), input_matrix.dtype).at[:n].set(input_matrix)
    out = _out_matmul(x_pad, agg, W)
    return out[:n]


# speedup vs baseline: 1.4336x; 1.4336x over previous
import functools

import jax
import jax.numpy as jnp
from jax import lax
from jax.experimental import pallas as pl
from jax.experimental.pallas import tpu as pltpu
from jax.experimental.pallas import tpu_sc as plsc

_LANES = 128
_R = 512


def _round_up(x, m):
    return (x + m - 1) // m * m


def _num_workers():
    info = plsc.get_sparse_core_info()
    return info.num_cores, info.num_subcores


def _sc_gather(table, idx):
    e_pad = idx.shape[0]
    d = table.shape[1]
    nc, ns = _num_workers()
    nw = nc * ns
    per_w = e_pad // nw
    assert e_pad % (nw * _LANES) == 0
    n_ch = per_w // _LANES
    mesh = plsc.VectorSubcoreMesh(core_axis_name="c", subcore_axis_name="s")

    @functools.partial(
        pl.kernel,
        mesh=mesh,
        out_type=jax.ShapeDtypeStruct((e_pad, d), jnp.float32),
        scratch_types=[
            pltpu.VMEM((_LANES,), jnp.int32),
            pltpu.VMEM((_LANES, d), jnp.float32),
            pltpu.SemaphoreType.DMA,
        ],
    )
    def k(table_hbm, idx_hbm, out_hbm, idx_v, rows_v, sem):
        wid = lax.axis_index("s") * nc + lax.axis_index("c")
        base = wid * per_w

        def body(i, carry):
            start = base + i * _LANES
            pltpu.sync_copy(idx_hbm.at[pl.ds(start, _LANES)], idx_v)
            pltpu.async_copy(table_hbm.at[idx_v], rows_v, sem).wait()
            pltpu.sync_copy(rows_v, out_hbm.at[pl.ds(start, _LANES)])
            return carry

        lax.fori_loop(0, n_ch, body, 0)

    return k(table, idx)


def _sc_scatter(rows, idx3):
    n_pad, d = rows.shape
    nw, kk, _ = idx3.shape
    per_w = kk * _LANES
    nc, ns = _num_workers()
    assert nw == nc * ns and n_pad == nw * per_w
    mesh = plsc.VectorSubcoreMesh(core_axis_name="c", subcore_axis_name="s")

    @functools.partial(
        pl.kernel,
        mesh=mesh,
        out_type=jax.ShapeDtypeStruct((n_pad, d), jnp.float32),
        scratch_types=[
            pltpu.VMEM((kk, _LANES), jnp.int32),
            pltpu.VMEM((_LANES, d), jnp.float32),
            pltpu.SemaphoreType.DMA,
        ],
    )
    def k(rows_hbm, idx_hbm, out_hbm, idx_v, buf_v, sem):
        wid = lax.axis_index("s") * nc + lax.axis_index("c")
        pltpu.sync_copy(idx_hbm.at[wid], idx_v)
        for c in range(kk):
            pltpu.sync_copy(
                rows_hbm.at[pl.ds(wid * per_w + c * _LANES, _LANES)], buf_v)
            pltpu.async_copy(buf_v, out_hbm.at[idx_v.at[c]], sem).wait()

    return k(rows, idx3)


def _lstm_packed(deg_sorted, x_tm, wih_t, whh_t, bias, n_pad, interpret=False):
    e_pad, d = x_tm.shape
    h = whh_t.shape[0]

    def body(deg_ref, x_hbm, wih_ref, whh_ref, b_ref, h_ref, c_ref,
             xbuf0, xbuf1, sem0, sem1):
        h_ref[...] = jnp.zeros_like(h_ref)
        c_ref[...] = jnp.zeros_like(c_ref)
        max_deg = deg_ref[0]

        def issue(sl, start):
            def go(buf, sm):
                pltpu.make_async_copy(
                    x_hbm.at[pl.ds(start, _R)], buf, sm).start()
            pl.when(sl == 0)(lambda: go(xbuf0, sem0))
            pl.when(sl != 0)(lambda: go(xbuf1, sem1))

        def wait_slot(sl):
            def go(buf, sm):
                pltpu.make_async_copy(
                    x_hbm.at[pl.ds(0, _R)], buf, sm).wait()
            pl.when(sl == 0)(lambda: go(xbuf0, sem0))
            pl.when(sl != 0)(lambda: go(xbuf1, sem1))

        issue(jnp.int32(0), jnp.int32(0))

        def t_cond(s):
            return s[0] < max_deg

        def t_body(s):
            t, ptr, cnt, k = s

            def c_cond(c):
                return jnp.logical_and(c > 0, deg_ref[c - 1] <= t)

            cnt = lax.while_loop(c_cond, lambda c: c - 1, cnt)
            nch = (cnt + (_R - 1)) // _R

            def chunk(ci, k):
                row0 = ci * _R
                nxt = jnp.where(ci + 1 < nch, ptr + row0 + _R, ptr + cnt)
                issue((k + 1) % 2, nxt)
                wait_slot(k % 2)
                x = lax.cond(k % 2 == 0, lambda: xbuf0[...],
                             lambda: xbuf1[...])
                hs = h_ref[pl.ds(row0, _R), :]
                cs = c_ref[pl.ds(row0, _R), :]
                g = jnp.dot(x, wih_ref[...], preferred_element_type=jnp.float32)
                g = g + jnp.dot(hs, whh_ref[...],
                                preferred_element_type=jnp.float32)
                g = g + b_ref[...]
                gi = jax.nn.sigmoid(g[:, :h])
                gf = jax.nn.sigmoid(g[:, h:2 * h])
                gg = jnp.tanh(g[:, 2 * h:3 * h])
                go = jax.nn.sigmoid(g[:, 3 * h:])
                c_new = gf * cs + gi * gg
                h_new = go * jnp.tanh(c_new)
                m = (row0 + lax.broadcasted_iota(jnp.int32, (_R, 1), 0)) < cnt
                h_ref[pl.ds(row0, _R), :] = jnp.where(m, h_new, hs)
                c_ref[pl.ds(row0, _R), :] = jnp.where(m, c_new, cs)
                return k + 1

            k = lax.fori_loop(0, nch, chunk, k)
            return (t + 1, ptr + cnt, cnt, k)

        s_fin = lax.while_loop(
            t_cond, t_body,
            (jnp.int32(0), jnp.int32(0), jnp.int32(n_pad), jnp.int32(0)))
        wait_slot(s_fin[3] % 2)

    return pl.pallas_call(
        body,
        in_specs=[
            pl.BlockSpec(memory_space=pltpu.SMEM),
            pl.BlockSpec(memory_space=pl.ANY),
            pl.BlockSpec(memory_space=pltpu.VMEM),
            pl.BlockSpec(memory_space=pltpu.VMEM),
            pl.BlockSpec(memory_space=pltpu.VMEM),
        ],
        out_specs=pl.BlockSpec(memory_space=pltpu.VMEM),
        out_shape=jax.ShapeDtypeStruct((n_pad, h), jnp.float32),
        scratch_shapes=[
            pltpu.VMEM((n_pad, h), jnp.float32),
            pltpu.VMEM((_R, d), jnp.float32),
            pltpu.VMEM((_R, d), jnp.float32),
            pltpu.SemaphoreType.DMA,
            pltpu.SemaphoreType.DMA,
        ],
        interpret=interpret,
    )(deg_sorted, x_tm, wih_t, whh_t, bias)


def _out_matmul(x_pad, agg_pad, w, interpret=False):
    n_pad, d = x_pad.shape
    h = agg_pad.shape[1]
    out_f = w.shape[1]
    blk = 256

    def body(x_ref, a_ref, w_ref, o_ref):
        o_ref[...] = (
            jnp.dot(x_ref[...], w_ref[:d, :], preferred_element_type=jnp.float32)
            + jnp.dot(a_ref[...], w_ref[d:, :],
                      preferred_element_type=jnp.float32))

    return pl.pallas_call(
        body,
        grid=(n_pad // blk,),
        in_specs=[
            pl.BlockSpec((blk, d), lambda i: (i, 0)),
            pl.BlockSpec((blk, h), lambda i: (i, 0)),
            pl.BlockSpec((d + h, out_f), lambda i: (0, 0)),
        ],
        out_specs=pl.BlockSpec((blk, out_f), lambda i: (i, 0)),
        out_shape=jax.ShapeDtypeStruct((n_pad, out_f), jnp.float32),
        interpret=interpret,
    )(x_pad, agg_pad, w)


def kernel(input_matrix, W, Wih, Whh, bih, bhh, edge_index):
    n, d = input_matrix.shape
    h = Whh.shape[1]
    e = edge_index.shape[1]
    src = edge_index[0]
    dst = edge_index[1]

    nc, ns = _num_workers()
    nw = nc * ns
    lane_blk = nw * _LANES
    n_pad = _round_up(max(n, _R), lane_blk)
    e_pad = _round_up(e + _R, lane_blk)

    src_s, dst_s = lax.sort((src, dst), num_keys=1, is_stable=True)
    ar = jnp.arange(e, dtype=jnp.int32)
    brk = src_s[1:] != src_s[:-1]
    is_start = jnp.concatenate([jnp.ones((1,), bool), brk])
    is_last = jnp.concatenate([brk, jnp.ones((1,), bool)])
    seg_start = lax.cummax(jnp.where(is_start, ar, 0))
    seg_last = jnp.flip(lax.cummin(jnp.flip(jnp.where(is_last, ar, e - 1))))
    t_j = ar - seg_start
    negdeg_e = seg_start - seg_last - 1
    _, _, _, dst_packed = lax.sort(
        (t_j, negdeg_e, src_s, dst_s), num_keys=3, is_stable=False)
    gather_idx = jnp.concatenate(
        [dst_packed, jnp.zeros((e_pad - e,), jnp.int32)])

    deg = jnp.bincount(src, length=n).astype(jnp.int32)
    negdeg_n, node_order = lax.sort(
        (-deg, jnp.arange(n, dtype=jnp.int32)), num_keys=1, is_stable=True)
    deg_pad = jnp.zeros((n_pad,), jnp.int32).at[:n].set(-negdeg_n)

    x_tm = _sc_gather(input_matrix, gather_idx)

    wih_t = Wih.T
    whh_t = Whh.T
    bias = (bih + bhh).reshape(1, -1)
    h_fin = _lstm_packed(deg_pad, x_tm, wih_t, whh_t, bias, n_pad)

    scat = jnp.concatenate(
        [node_order, jnp.arange(n, n_pad, dtype=jnp.int32)]
    ).reshape(nw, -1, _LANES)
    agg = _sc_scatter(h_fin, scat)

    x_pad = jnp.zeros((n_pad, d), input_matrix.dtype).at[:n].set(input_matrix)
    out = _out_matmul(x_pad, agg, W)
    return out[:n]


# speedup vs baseline: 1.6172x; 1.1281x over previous
import functools

import jax
import jax.numpy as jnp
from jax import lax
from jax.experimental import pallas as pl
from jax.experimental.pallas import tpu as pltpu
from jax.experimental.pallas import tpu_sc as plsc

_LANES = 128
_R = 512


def _round_up(x, m):
    return (x + m - 1) // m * m


def _num_workers():
    info = plsc.get_sparse_core_info()
    return info.num_cores, info.num_subcores


def _sc_gather(table, idx):
    e_pad = idx.shape[0]
    d = table.shape[1]
    nc, ns = _num_workers()
    nw = nc * ns
    per_w = e_pad // nw
    assert e_pad % (nw * _LANES) == 0
    n_ch = per_w // _LANES
    mesh = plsc.VectorSubcoreMesh(core_axis_name="c", subcore_axis_name="s")

    @functools.partial(
        pl.kernel,
        mesh=mesh,
        out_type=jax.ShapeDtypeStruct((e_pad, d), jnp.float32),
        scratch_types=[
            pltpu.VMEM((per_w,), jnp.int32),
            pltpu.VMEM((_LANES, d), jnp.float32),
            pltpu.VMEM((_LANES, d), jnp.float32),
            pltpu.SemaphoreType.DMA,
            pltpu.SemaphoreType.DMA,
            pltpu.SemaphoreType.DMA,
            pltpu.SemaphoreType.DMA,
        ],
    )
    def k(table_hbm, idx_hbm, out_hbm, idx_v, rows0, rows1,
          gsem0, gsem1, wsem0, wsem1):
        wid = lax.axis_index("s") * nc + lax.axis_index("c")
        base = wid * per_w
        pltpu.sync_copy(idx_hbm.at[pl.ds(base, per_w)], idx_v)

        def gstart(c, sl):
            def go(buf, sm):
                pltpu.async_copy(
                    table_hbm.at[idx_v.at[pl.ds(c * _LANES, _LANES)]], buf, sm)
            pl.when(sl == 0)(lambda: go(rows0, gsem0))
            pl.when(sl != 0)(lambda: go(rows1, gsem1))

        def gwait(sl):
            def go(buf, sm):
                pltpu.make_async_copy(
                    table_hbm.at[pl.ds(0, _LANES)], buf, sm).wait()
            pl.when(sl == 0)(lambda: go(rows0, gsem0))
            pl.when(sl != 0)(lambda: go(rows1, gsem1))

        def wstart(c, sl):
            def go(buf, sm):
                pltpu.async_copy(
                    buf, out_hbm.at[pl.ds(base + c * _LANES, _LANES)], sm)
            pl.when(sl == 0)(lambda: go(rows0, wsem0))
            pl.when(sl != 0)(lambda: go(rows1, wsem1))

        def wwait(sl):
            def go(buf, sm):
                pltpu.make_async_copy(
                    table_hbm.at[pl.ds(0, _LANES)], buf, sm).wait()
            pl.when(sl == 0)(lambda: go(rows0, wsem0))
            pl.when(sl != 0)(lambda: go(rows1, wsem1))

        gstart(jnp.int32(0), jnp.int32(0))

        def body(c, carry):
            sl = c % 2
            pl.when(c >= 1)(lambda: wwait(1 - sl))
            pl.when(c + 1 < n_ch)(lambda: gstart(c + 1, 1 - sl))
            gwait(sl)
            wstart(c, sl)
            return carry

        lax.fori_loop(0, n_ch, body, 0)
        wwait((n_ch - 1) % 2)

    return k(table, idx)


def _sc_scatter(rows, idx3):
    n_pad, d = rows.shape
    nw, kk, _ = idx3.shape
    per_w = kk * _LANES
    nc, ns = _num_workers()
    assert nw == nc * ns and n_pad == nw * per_w
    mesh = plsc.VectorSubcoreMesh(core_axis_name="c", subcore_axis_name="s")

    @functools.partial(
        pl.kernel,
        mesh=mesh,
        out_type=jax.ShapeDtypeStruct((n_pad, d), jnp.float32),
        scratch_types=[
            pltpu.VMEM((kk, _LANES), jnp.int32),
            pltpu.VMEM((_LANES, d), jnp.float32),
            pltpu.SemaphoreType.DMA,
        ],
    )
    def k(rows_hbm, idx_hbm, out_hbm, idx_v, buf_v, sem):
        wid = lax.axis_index("s") * nc + lax.axis_index("c")
        pltpu.sync_copy(idx_hbm.at[wid], idx_v)
        for c in range(kk):
            pltpu.sync_copy(
                rows_hbm.at[pl.ds(wid * per_w + c * _LANES, _LANES)], buf_v)
            pltpu.async_copy(buf_v, out_hbm.at[idx_v.at[c]], sem).wait()

    return k(rows, idx3)


def _lstm_packed(deg_sorted, x_tm, wih_t, whh_t, bias, n_pad, interpret=False):
    e_pad, d = x_tm.shape
    h = whh_t.shape[0]

    def body(deg_ref, x_hbm, wih_ref, whh_ref, b_ref, h_ref, c_ref,
             xbuf0, xbuf1, sem0, sem1):
        h_ref[...] = jnp.zeros_like(h_ref)
        c_ref[...] = jnp.zeros_like(c_ref)
        max_deg = deg_ref[0]

        def issue(sl, start):
            def go(buf, sm):
                pltpu.make_async_copy(
                    x_hbm.at[pl.ds(start, _R)], buf, sm).start()
            pl.when(sl == 0)(lambda: go(xbuf0, sem0))
            pl.when(sl != 0)(lambda: go(xbuf1, sem1))

        def wait_slot(sl):
            def go(buf, sm):
                pltpu.make_async_copy(
                    x_hbm.at[pl.ds(0, _R)], buf, sm).wait()
            pl.when(sl == 0)(lambda: go(xbuf0, sem0))
            pl.when(sl != 0)(lambda: go(xbuf1, sem1))

        issue(jnp.int32(0), jnp.int32(0))

        def t_cond(s):
            return s[0] < max_deg

        def t_body(s):
            t, ptr, cnt, k = s

            def c_cond(c):
                return jnp.logical_and(c > 0, deg_ref[c - 1] <= t)

            cnt = lax.while_loop(c_cond, lambda c: c - 1, cnt)
            nch = (cnt + (_R - 1)) // _R

            def chunk(ci, k):
                row0 = ci * _R
                nxt = jnp.where(ci + 1 < nch, ptr + row0 + _R, ptr + cnt)
                issue((k + 1) % 2, nxt)
                wait_slot(k % 2)
                x = lax.cond(k % 2 == 0, lambda: xbuf0[...],
                             lambda: xbuf1[...])
                hs = h_ref[pl.ds(row0, _R), :]
                cs = c_ref[pl.ds(row0, _R), :]
                g = jnp.dot(x, wih_ref[...], preferred_element_type=jnp.float32)
                g = g + jnp.dot(hs, whh_ref[...],
                                preferred_element_type=jnp.float32)
                g = g + b_ref[...]
                gi = jax.nn.sigmoid(g[:, :h])
                gf = jax.nn.sigmoid(g[:, h:2 * h])
                gg = jnp.tanh(g[:, 2 * h:3 * h])
                go = jax.nn.sigmoid(g[:, 3 * h:])
                c_new = gf * cs + gi * gg
                h_new = go * jnp.tanh(c_new)
                m = (row0 + lax.broadcasted_iota(jnp.int32, (_R, 1), 0)) < cnt
                h_ref[pl.ds(row0, _R), :] = jnp.where(m, h_new, hs)
                c_ref[pl.ds(row0, _R), :] = jnp.where(m, c_new, cs)
                return k + 1

            k = lax.fori_loop(0, nch, chunk, k)
            return (t + 1, ptr + cnt, cnt, k)

        s_fin = lax.while_loop(
            t_cond, t_body,
            (jnp.int32(0), jnp.int32(0), jnp.int32(n_pad), jnp.int32(0)))
        wait_slot(s_fin[3] % 2)

    return pl.pallas_call(
        body,
        in_specs=[
            pl.BlockSpec(memory_space=pltpu.SMEM),
            pl.BlockSpec(memory_space=pl.ANY),
            pl.BlockSpec(memory_space=pltpu.VMEM),
            pl.BlockSpec(memory_space=pltpu.VMEM),
            pl.BlockSpec(memory_space=pltpu.VMEM),
        ],
        out_specs=pl.BlockSpec(memory_space=pltpu.VMEM),
        out_shape=jax.ShapeDtypeStruct((n_pad, h), jnp.float32),
        scratch_shapes=[
            pltpu.VMEM((n_pad, h), jnp.float32),
            pltpu.VMEM((_R, d), jnp.float32),
            pltpu.VMEM((_R, d), jnp.float32),
            pltpu.SemaphoreType.DMA,
            pltpu.SemaphoreType.DMA,
        ],
        interpret=interpret,
    )(deg_sorted, x_tm, wih_t, whh_t, bias)


def _out_matmul(x_pad, agg_pad, w, interpret=False):
    n_pad, d = x_pad.shape
    h = agg_pad.shape[1]
    out_f = w.shape[1]
    blk = 256

    def body(x_ref, a_ref, w_ref, o_ref):
        o_ref[...] = (
            jnp.dot(x_ref[...], w_ref[:d, :], preferred_element_type=jnp.float32)
            + jnp.dot(a_ref[...], w_ref[d:, :],
                      preferred_element_type=jnp.float32))

    return pl.pallas_call(
        body,
        grid=(n_pad // blk,),
        in_specs=[
            pl.BlockSpec((blk, d), lambda i: (i, 0)),
            pl.BlockSpec((blk, h), lambda i: (i, 0)),
            pl.BlockSpec((d + h, out_f), lambda i: (0, 0)),
        ],
        out_specs=pl.BlockSpec((blk, out_f), lambda i: (i, 0)),
        out_shape=jax.ShapeDtypeStruct((n_pad, out_f), jnp.float32),
        interpret=interpret,
    )(x_pad, agg_pad, w)


def kernel(input_matrix, W, Wih, Whh, bih, bhh, edge_index):
    n, d = input_matrix.shape
    h = Whh.shape[1]
    e = edge_index.shape[1]
    src = edge_index[0]
    dst = edge_index[1]

    nc, ns = _num_workers()
    nw = nc * ns
    lane_blk = nw * _LANES
    n_pad = _round_up(max(n, _R), lane_blk)
    e_pad = _round_up(e + _R, lane_blk)

    src_s, dst_s = lax.sort((src, dst), num_keys=1, is_stable=True)
    ar = jnp.arange(e, dtype=jnp.int32)
    brk = src_s[1:] != src_s[:-1]
    is_start = jnp.concatenate([jnp.ones((1,), bool), brk])
    is_last = jnp.concatenate([brk, jnp.ones((1,), bool)])
    seg_start = lax.cummax(jnp.where(is_start, ar, 0))
    seg_last = jnp.flip(lax.cummin(jnp.flip(jnp.where(is_last, ar, e - 1))))
    t_j = ar - seg_start
    negdeg_e = seg_start - seg_last - 1
    if e < (1 << 18) and n <= (1 << 14):
        key2 = (((jnp.uint32(e) + negdeg_e.astype(jnp.uint32))
                 << jnp.uint32(14)) | src_s.astype(jnp.uint32))
        _, _, dst_packed = lax.sort(
            (t_j, key2, dst_s), num_keys=2, is_stable=False)
    else:
        _, _, _, dst_packed = lax.sort(
            (t_j, negdeg_e, src_s, dst_s), num_keys=3, is_stable=False)
    gather_idx = jnp.concatenate(
        [dst_packed, jnp.zeros((e_pad - e,), jnp.int32)])

    deg = jnp.bincount(src, length=n).astype(jnp.int32)
    negdeg_n, node_order = lax.sort(
        (-deg, jnp.arange(n, dtype=jnp.int32)), num_keys=1, is_stable=True)
    deg_pad = jnp.zeros((n_pad,), jnp.int32).at[:n].set(-negdeg_n)

    x_tm = _sc_gather(input_matrix, gather_idx)

    wih_t = Wih.T
    whh_t = Whh.T
    bias = (bih + bhh).reshape(1, -1)
    h_fin = _lstm_packed(deg_pad, x_tm, wih_t, whh_t, bias, n_pad)

    scat = jnp.concatenate(
        [node_order, jnp.arange(n, n_pad, dtype=jnp.int32)]
    ).reshape(nw, -1, _LANES)
    agg = _sc_scatter(h_fin, scat)

    x_pad = jnp.zeros((n_pad, d), input_matrix.dtype).at[:n].set(input_matrix)
    out = _out_matmul(x_pad, agg, W)
    return out[:n]


# speedup vs baseline: 1.7035x; 1.0533x over previous
import functools

import jax
import jax.numpy as jnp
from jax import lax
from jax.experimental import pallas as pl
from jax.experimental.pallas import tpu as pltpu
from jax.experimental.pallas import tpu_sc as plsc

_LANES = 128
_R = 512


def _round_up(x, m):
    return (x + m - 1) // m * m


def _num_workers():
    info = plsc.get_sparse_core_info()
    return info.num_cores, info.num_subcores


def _sc_gather(table, idx):
    e_pad = idx.shape[0]
    d = table.shape[1]
    nc, ns = _num_workers()
    nw = nc * ns
    per_w = e_pad // nw
    assert e_pad % (nw * _LANES) == 0
    n_ch = per_w // _LANES
    mesh = plsc.VectorSubcoreMesh(core_axis_name="c", subcore_axis_name="s")

    nb = 6

    @functools.partial(
        pl.kernel,
        mesh=mesh,
        out_type=jax.ShapeDtypeStruct((e_pad, d), jnp.float32),
        scratch_types=(
            [pltpu.VMEM((per_w,), jnp.int32)]
            + [pltpu.VMEM((_LANES, d), jnp.float32)] * nb
            + [pltpu.SemaphoreType.DMA] * (2 * nb)
        ),
    )
    def k(table_hbm, idx_hbm, out_hbm, idx_v, *bufs_and_sems):
        bufs = bufs_and_sems[:nb]
        gsems = bufs_and_sems[nb:2 * nb]
        wsems = bufs_and_sems[2 * nb:3 * nb]
        wid = lax.axis_index("s") * nc + lax.axis_index("c")
        base = wid * per_w
        pltpu.sync_copy(idx_hbm.at[pl.ds(base, per_w)], idx_v)

        def _sel(sl, go):
            for i in range(nb):
                pl.when(sl == i)(functools.partial(go, bufs[i], gsems[i],
                                                   wsems[i]))

        def gstart(c, sl):
            def go(buf, gsm, wsm):
                pltpu.async_copy(
                    table_hbm.at[idx_v.at[pl.ds(c * _LANES, _LANES)]],
                    buf, gsm)
            _sel(sl, go)

        def gwait(sl):
            def go(buf, gsm, wsm):
                pltpu.make_async_copy(
                    table_hbm.at[pl.ds(0, _LANES)], buf, gsm).wait()
            _sel(sl, go)

        def wstart(c, sl):
            def go(buf, gsm, wsm):
                pltpu.async_copy(
                    buf, out_hbm.at[pl.ds(base + c * _LANES, _LANES)], wsm)
            _sel(sl, go)

        def wwait(sl):
            def go(buf, gsm, wsm):
                pltpu.make_async_copy(
                    table_hbm.at[pl.ds(0, _LANES)], buf, wsm).wait()
            _sel(sl, go)

        for i in range(min(3, n_ch)):
            gstart(jnp.int32(i), jnp.int32(i))

        def body(c, carry):
            sl = c % nb
            pl.when(c >= 3)(lambda: wwait((c - 3) % nb))
            pl.when(c + 3 < n_ch)(lambda: gstart(c + 3, (c + 3) % nb))
            gwait(sl)
            wstart(c, sl)
            return carry

        lax.fori_loop(0, n_ch, body, 0)
        for cc in range(max(0, n_ch - 3), n_ch):
            wwait(jnp.int32(cc % nb))

    return k(table, idx)


def _sc_scatter(rows, idx3):
    n_pad, d = rows.shape
    nw, kk, _ = idx3.shape
    per_w = kk * _LANES
    nc, ns = _num_workers()
    assert nw == nc * ns and n_pad == nw * per_w
    mesh = plsc.VectorSubcoreMesh(core_axis_name="c", subcore_axis_name="s")

    @functools.partial(
        pl.kernel,
        mesh=mesh,
        out_type=jax.ShapeDtypeStruct((n_pad, d), jnp.float32),
        scratch_types=[
            pltpu.VMEM((kk, _LANES), jnp.int32),
            pltpu.VMEM((_LANES, d), jnp.float32),
            pltpu.SemaphoreType.DMA,
        ],
    )
    def k(rows_hbm, idx_hbm, out_hbm, idx_v, buf_v, sem):
        wid = lax.axis_index("s") * nc + lax.axis_index("c")
        pltpu.sync_copy(idx_hbm.at[wid], idx_v)
        for c in range(kk):
            pltpu.sync_copy(
                rows_hbm.at[pl.ds(wid * per_w + c * _LANES, _LANES)], buf_v)
            pltpu.async_copy(buf_v, out_hbm.at[idx_v.at[c]], sem).wait()

    return k(rows, idx3)


def _lstm_packed(deg_sorted, x_tm, wih_t, whh_t, bias, n_pad, interpret=False):
    e_pad, d = x_tm.shape
    h = whh_t.shape[0]

    def body(deg_ref, x_hbm, wih_ref, whh_ref, b_ref, h_ref, c_ref,
             xbuf0, xbuf1, sem0, sem1):
        h_ref[...] = jnp.zeros_like(h_ref)
        c_ref[...] = jnp.zeros_like(c_ref)
        max_deg = deg_ref[0]

        def issue(sl, start):
            def go(buf, sm):
                pltpu.make_async_copy(
                    x_hbm.at[pl.ds(start, _R)], buf, sm).start()
            pl.when(sl == 0)(lambda: go(xbuf0, sem0))
            pl.when(sl != 0)(lambda: go(xbuf1, sem1))

        def wait_slot(sl):
            def go(buf, sm):
                pltpu.make_async_copy(
                    x_hbm.at[pl.ds(0, _R)], buf, sm).wait()
            pl.when(sl == 0)(lambda: go(xbuf0, sem0))
            pl.when(sl != 0)(lambda: go(xbuf1, sem1))

        issue(jnp.int32(0), jnp.int32(0))

        def t_cond(s):
            return s[0] < max_deg

        def t_body(s):
            t, ptr, cnt, k = s

            def c_cond(c):
                return jnp.logical_and(c > 0, deg_ref[c - 1] <= t)

            cnt = lax.while_loop(c_cond, lambda c: c - 1, cnt)
            nch = (cnt + (_R - 1)) // _R

            def chunk(ci, k):
                row0 = ci * _R
                nxt = jnp.where(ci + 1 < nch, ptr + row0 + _R, ptr + cnt)
                issue((k + 1) % 2, nxt)
                wait_slot(k % 2)
                x = lax.cond(k % 2 == 0, lambda: xbuf0[...],
                             lambda: xbuf1[...])
                hs = h_ref[pl.ds(row0, _R), :]
                cs = c_ref[pl.ds(row0, _R), :]
                g = jnp.dot(x, wih_ref[...], preferred_element_type=jnp.float32)
                g = g + jnp.dot(hs, whh_ref[...],
                                preferred_element_type=jnp.float32)
                g = g + b_ref[...]
                gi = jax.nn.sigmoid(g[:, :h])
                gf = jax.nn.sigmoid(g[:, h:2 * h])
                gg = jnp.tanh(g[:, 2 * h:3 * h])
                go = jax.nn.sigmoid(g[:, 3 * h:])
                c_new = gf * cs + gi * gg
                h_new = go * jnp.tanh(c_new)
                m = (row0 + lax.broadcasted_iota(jnp.int32, (_R, 1), 0)) < cnt
                h_ref[pl.ds(row0, _R), :] = jnp.where(m, h_new, hs)
                c_ref[pl.ds(row0, _R), :] = jnp.where(m, c_new, cs)
                return k + 1

            k = lax.fori_loop(0, nch, chunk, k)
            return (t + 1, ptr + cnt, cnt, k)

        s_fin = lax.while_loop(
            t_cond, t_body,
            (jnp.int32(0), jnp.int32(0), jnp.int32(n_pad), jnp.int32(0)))
        wait_slot(s_fin[3] % 2)

    return pl.pallas_call(
        body,
        in_specs=[
            pl.BlockSpec(memory_space=pltpu.SMEM),
            pl.BlockSpec(memory_space=pl.ANY),
            pl.BlockSpec(memory_space=pltpu.VMEM),
            pl.BlockSpec(memory_space=pltpu.VMEM),
            pl.BlockSpec(memory_space=pltpu.VMEM),
        ],
        out_specs=pl.BlockSpec(memory_space=pltpu.VMEM),
        out_shape=jax.ShapeDtypeStruct((n_pad, h), jnp.float32),
        scratch_shapes=[
            pltpu.VMEM((n_pad, h), jnp.float32),
            pltpu.VMEM((_R, d), jnp.float32),
            pltpu.VMEM((_R, d), jnp.float32),
            pltpu.SemaphoreType.DMA,
            pltpu.SemaphoreType.DMA,
        ],
        interpret=interpret,
    )(deg_sorted, x_tm, wih_t, whh_t, bias)


def _out_matmul(x_pad, agg_pad, w, interpret=False):
    n_pad, d = x_pad.shape
    h = agg_pad.shape[1]
    out_f = w.shape[1]
    blk = 256

    def body(x_ref, a_ref, w_ref, o_ref):
        o_ref[...] = (
            jnp.dot(x_ref[...], w_ref[:d, :], preferred_element_type=jnp.float32)
            + jnp.dot(a_ref[...], w_ref[d:, :],
                      preferred_element_type=jnp.float32))

    return pl.pallas_call(
        body,
        grid=(n_pad // blk,),
        in_specs=[
            pl.BlockSpec((blk, d), lambda i: (i, 0)),
            pl.BlockSpec((blk, h), lambda i: (i, 0)),
            pl.BlockSpec((d + h, out_f), lambda i: (0, 0)),
        ],
        out_specs=pl.BlockSpec((blk, out_f), lambda i: (i, 0)),
        out_shape=jax.ShapeDtypeStruct((n_pad, out_f), jnp.float32),
        interpret=interpret,
    )(x_pad, agg_pad, w)


def kernel(input_matrix, W, Wih, Whh, bih, bhh, edge_index):
    n, d = input_matrix.shape
    h = Whh.shape[1]
    e = edge_index.shape[1]
    src = edge_index[0]
    dst = edge_index[1]

    nc, ns = _num_workers()
    nw = nc * ns
    lane_blk = nw * _LANES
    n_pad = _round_up(max(n, _R), lane_blk)
    e_pad = _round_up(e + _R, lane_blk)

    fast_keys = e <= (1 << 18) and n <= (1 << 14)
    if fast_keys:
        key1 = ((src.astype(jnp.uint32) << jnp.uint32(18))
                | jnp.arange(e, dtype=jnp.uint32))
        key1_s, dst_s = lax.sort((key1, dst), num_keys=1, is_stable=False)
        src_s = (key1_s >> jnp.uint32(18)).astype(jnp.int32)
    else:
        src_s, dst_s = lax.sort((src, dst), num_keys=1, is_stable=True)
    ar = jnp.arange(e, dtype=jnp.int32)
    brk = src_s[1:] != src_s[:-1]
    is_start = jnp.concatenate([jnp.ones((1,), bool), brk])
    is_last = jnp.concatenate([brk, jnp.ones((1,), bool)])
    seg_start = lax.cummax(jnp.where(is_start, ar, 0))
    seg_last = jnp.flip(lax.cummin(jnp.flip(jnp.where(is_last, ar, e - 1))))
    t_j = ar - seg_start
    negdeg_e = seg_start - seg_last - 1
    if e < (1 << 18) and n <= (1 << 14):
        key2 = (((jnp.uint32(e) + negdeg_e.astype(jnp.uint32))
                 << jnp.uint32(14)) | src_s.astype(jnp.uint32))
        _, _, dst_packed = lax.sort(
            (t_j, key2, dst_s), num_keys=2, is_stable=False)
    else:
        _, _, _, dst_packed = lax.sort(
            (t_j, negdeg_e, src_s, dst_s), num_keys=3, is_stable=False)
    gather_idx = jnp.concatenate(
        [dst_packed, jnp.zeros((e_pad - e,), jnp.int32)])

    deg = jnp.bincount(src, length=n).astype(jnp.int32)
    if e < (1 << 18) and n <= (1 << 14):
        keyn = (((jnp.uint32(e) - deg.astype(jnp.uint32)) << jnp.uint32(14))
                | jnp.arange(n, dtype=jnp.uint32))
        keyn_s = lax.sort(keyn, is_stable=False)
        node_order = (keyn_s & jnp.uint32((1 << 14) - 1)).astype(jnp.int32)
        deg_desc = (jnp.uint32(e) - (keyn_s >> jnp.uint32(14))).astype(jnp.int32)
    else:
        negdeg_n, node_order = lax.sort(
            (-deg, jnp.arange(n, dtype=jnp.int32)), num_keys=1, is_stable=True)
        deg_desc = -negdeg_n
    deg_pad = jnp.zeros((n_pad,), jnp.int32).at[:n].set(deg_desc)

    x_tm = _sc_gather(input_matrix, gather_idx)

    wih_t = Wih.T
    whh_t = Whh.T
    bias = (bih + bhh).reshape(1, -1)
    h_fin = _lstm_packed(deg_pad, x_tm, wih_t, whh_t, bias, n_pad)

    scat = jnp.concatenate(
        [node_order, jnp.arange(n, n_pad, dtype=jnp.int32)]
    ).reshape(nw, -1, _LANES)
    agg = _sc_scatter(h_fin, scat)

    x_pad = jnp.zeros((n_pad, d), input_matrix.dtype).at[:n].set(input_matrix)
    out = _out_matmul(x_pad, agg, W)
    return out[:n]


# speedup vs baseline: 1.7353x; 1.0187x over previous
import functools

import jax
import jax.numpy as jnp
from jax import lax
from jax.experimental import pallas as pl
from jax.experimental.pallas import tpu as pltpu
from jax.experimental.pallas import tpu_sc as plsc

_LANES = 128
_R = 512


def _round_up(x, m):
    return (x + m - 1) // m * m


def _num_workers():
    info = plsc.get_sparse_core_info()
    return info.num_cores, info.num_subcores


def _sc_gather(table, idx):
    e_pad = idx.shape[0]
    d = table.shape[1]
    nc, ns = _num_workers()
    nw = nc * ns
    per_w = e_pad // nw
    assert e_pad % (nw * _LANES) == 0
    n_ch = per_w // _LANES
    mesh = plsc.VectorSubcoreMesh(core_axis_name="c", subcore_axis_name="s")

    nb = 6

    @functools.partial(
        pl.kernel,
        mesh=mesh,
        out_type=jax.ShapeDtypeStruct((e_pad, d), jnp.float32),
        scratch_types=(
            [pltpu.VMEM((per_w,), jnp.int32)]
            + [pltpu.VMEM((_LANES, d), jnp.float32)] * nb
            + [pltpu.SemaphoreType.DMA] * (2 * nb)
        ),
    )
    def k(table_hbm, idx_hbm, out_hbm, idx_v, *bufs_and_sems):
        bufs = bufs_and_sems[:nb]
        gsems = bufs_and_sems[nb:2 * nb]
        wsems = bufs_and_sems[2 * nb:3 * nb]
        wid = lax.axis_index("s") * nc + lax.axis_index("c")
        base = wid * per_w
        pltpu.sync_copy(idx_hbm.at[pl.ds(base, per_w)], idx_v)

        def _sel(sl, go):
            for i in range(nb):
                pl.when(sl == i)(functools.partial(go, bufs[i], gsems[i],
                                                   wsems[i]))

        def gstart(c, sl):
            def go(buf, gsm, wsm):
                pltpu.async_copy(
                    table_hbm.at[idx_v.at[pl.ds(c * _LANES, _LANES)]],
                    buf, gsm)
            _sel(sl, go)

        def gwait(sl):
            def go(buf, gsm, wsm):
                pltpu.make_async_copy(
                    table_hbm.at[pl.ds(0, _LANES)], buf, gsm).wait()
            _sel(sl, go)

        def wstart(c, sl):
            def go(buf, gsm, wsm):
                pltpu.async_copy(
                    buf, out_hbm.at[pl.ds(base + c * _LANES, _LANES)], wsm)
            _sel(sl, go)

        def wwait(sl):
            def go(buf, gsm, wsm):
                pltpu.make_async_copy(
                    table_hbm.at[pl.ds(0, _LANES)], buf, wsm).wait()
            _sel(sl, go)

        for i in range(min(3, n_ch)):
            gstart(jnp.int32(i), jnp.int32(i))

        def body(c, carry):
            sl = c % nb
            pl.when(c >= 3)(lambda: wwait((c - 3) % nb))
            pl.when(c + 3 < n_ch)(lambda: gstart(c + 3, (c + 3) % nb))
            gwait(sl)
            wstart(c, sl)
            return carry

        lax.fori_loop(0, n_ch, body, 0)
        for cc in range(max(0, n_ch - 3), n_ch):
            wwait(jnp.int32(cc % nb))

    return k(table, idx)


def _sc_scatter(rows, idx3):
    n_pad, d = rows.shape
    nw, kk, _ = idx3.shape
    per_w = kk * _LANES
    nc, ns = _num_workers()
    assert nw == nc * ns and n_pad == nw * per_w
    mesh = plsc.VectorSubcoreMesh(core_axis_name="c", subcore_axis_name="s")

    @functools.partial(
        pl.kernel,
        mesh=mesh,
        out_type=jax.ShapeDtypeStruct((n_pad, d), jnp.float32),
        scratch_types=[
            pltpu.VMEM((kk, _LANES), jnp.int32),
            pltpu.VMEM((_LANES, d), jnp.float32),
            pltpu.SemaphoreType.DMA,
        ],
    )
    def k(rows_hbm, idx_hbm, out_hbm, idx_v, buf_v, sem):
        wid = lax.axis_index("s") * nc + lax.axis_index("c")
        pltpu.sync_copy(idx_hbm.at[wid], idx_v)
        for c in range(kk):
            pltpu.sync_copy(
                rows_hbm.at[pl.ds(wid * per_w + c * _LANES, _LANES)], buf_v)
            pltpu.async_copy(buf_v, out_hbm.at[idx_v.at[c]], sem).wait()

    return k(rows, idx3)


def _lstm_packed(deg_sorted, x_tm, wih_t, whh_t, bias, n_pad, interpret=False):
    e_pad, d = x_tm.shape
    h = whh_t.shape[0]

    def body(deg_ref, x_hbm, wih_ref, whh_ref, b_ref, h_ref, c_ref,
             xbuf0, xbuf1, sem0, sem1):
        h_ref[...] = jnp.zeros_like(h_ref)
        c_ref[...] = jnp.zeros_like(c_ref)
        max_deg = deg_ref[0]

        def issue(sl, start):
            def go(buf, sm):
                pltpu.make_async_copy(
                    x_hbm.at[pl.ds(start, _R)], buf, sm).start()
            pl.when(sl == 0)(lambda: go(xbuf0, sem0))
            pl.when(sl != 0)(lambda: go(xbuf1, sem1))

        def wait_slot(sl):
            def go(buf, sm):
                pltpu.make_async_copy(
                    x_hbm.at[pl.ds(0, _R)], buf, sm).wait()
            pl.when(sl == 0)(lambda: go(xbuf0, sem0))
            pl.when(sl != 0)(lambda: go(xbuf1, sem1))

        issue(jnp.int32(0), jnp.int32(0))

        def t_cond(s):
            return s[0] < max_deg

        def t_body(s):
            t, ptr, cnt, k = s

            def c_cond(c):
                return jnp.logical_and(c > 0, deg_ref[c - 1] <= t)

            cnt = lax.while_loop(c_cond, lambda c: c - 1, cnt)
            nch = (cnt + (_R - 1)) // _R

            def chunk(ci, k):
                row0 = ci * _R
                nxt = jnp.where(ci + 1 < nch, ptr + row0 + _R, ptr + cnt)
                issue((k + 1) % 2, nxt)
                wait_slot(k % 2)
                x = lax.cond(k % 2 == 0, lambda: xbuf0[...],
                             lambda: xbuf1[...])
                hs = h_ref[pl.ds(row0, _R), :]
                cs = c_ref[pl.ds(row0, _R), :]
                g = jnp.dot(x, wih_ref[...], preferred_element_type=jnp.float32)
                g = g + jnp.dot(hs, whh_ref[...],
                                preferred_element_type=jnp.float32)
                g = g + b_ref[...]
                gi = jax.nn.sigmoid(g[:, :h])
                gf = jax.nn.sigmoid(g[:, h:2 * h])
                gg = jnp.tanh(g[:, 2 * h:3 * h])
                go = jax.nn.sigmoid(g[:, 3 * h:])
                c_new = gf * cs + gi * gg
                h_new = go * jnp.tanh(c_new)
                m = (row0 + lax.broadcasted_iota(jnp.int32, (_R, 1), 0)) < cnt
                h_ref[pl.ds(row0, _R), :] = jnp.where(m, h_new, hs)
                c_ref[pl.ds(row0, _R), :] = jnp.where(m, c_new, cs)
                return k + 1

            k = lax.fori_loop(0, nch, chunk, k)
            return (t + 1, ptr + cnt, cnt, k)

        s_fin = lax.while_loop(
            t_cond, t_body,
            (jnp.int32(0), jnp.int32(0), jnp.int32(n_pad), jnp.int32(0)))
        wait_slot(s_fin[3] % 2)

    return pl.pallas_call(
        body,
        in_specs=[
            pl.BlockSpec(memory_space=pltpu.SMEM),
            pl.BlockSpec(memory_space=pl.ANY),
            pl.BlockSpec(memory_space=pltpu.VMEM),
            pl.BlockSpec(memory_space=pltpu.VMEM),
            pl.BlockSpec(memory_space=pltpu.VMEM),
        ],
        out_specs=pl.BlockSpec(memory_space=pltpu.VMEM),
        out_shape=jax.ShapeDtypeStruct((n_pad, h), jnp.float32),
        scratch_shapes=[
            pltpu.VMEM((n_pad, h), jnp.float32),
            pltpu.VMEM((_R, d), jnp.float32),
            pltpu.VMEM((_R, d), jnp.float32),
            pltpu.SemaphoreType.DMA,
            pltpu.SemaphoreType.DMA,
        ],
        interpret=interpret,
    )(deg_sorted, x_tm, wih_t, whh_t, bias)


def _out_matmul(x_rows, agg_pad, w, blk, interpret=False):
    n_rows, d = x_rows.shape
    h = agg_pad.shape[1]
    out_f = w.shape[1]

    def body(x_ref, a_ref, w_ref, o_ref):
        o_ref[...] = (
            jnp.dot(x_ref[...], w_ref[:d, :], preferred_element_type=jnp.float32)
            + jnp.dot(a_ref[...], w_ref[d:, :],
                      preferred_element_type=jnp.float32))

    return pl.pallas_call(
        body,
        grid=(n_rows // blk,),
        in_specs=[
            pl.BlockSpec((blk, d), lambda i: (i, 0)),
            pl.BlockSpec((blk, h), lambda i: (i, 0)),
            pl.BlockSpec((d + h, out_f), lambda i: (0, 0)),
        ],
        out_specs=pl.BlockSpec((blk, out_f), lambda i: (i, 0)),
        out_shape=jax.ShapeDtypeStruct((n_rows, out_f), jnp.float32),
        interpret=interpret,
    )(x_rows, agg_pad, w)


def kernel(input_matrix, W, Wih, Whh, bih, bhh, edge_index):
    n, d = input_matrix.shape
    h = Whh.shape[1]
    e = edge_index.shape[1]
    src = edge_index[0]
    dst = edge_index[1]

    nc, ns = _num_workers()
    nw = nc * ns
    lane_blk = nw * _LANES
    n_pad = _round_up(max(n, _R), lane_blk)
    e_pad = _round_up(e + _R, lane_blk)

    fast_keys = e <= (1 << 18) and n <= (1 << 14)
    if fast_keys:
        key1 = ((src.astype(jnp.uint32) << jnp.uint32(18))
                | jnp.arange(e, dtype=jnp.uint32))
        key1_s, dst_s = lax.sort((key1, dst), num_keys=1, is_stable=False)
        src_s = (key1_s >> jnp.uint32(18)).astype(jnp.int32)
    else:
        src_s, dst_s = lax.sort((src, dst), num_keys=1, is_stable=True)
    ar = jnp.arange(e, dtype=jnp.int32)
    brk = src_s[1:] != src_s[:-1]
    is_start = jnp.concatenate([jnp.ones((1,), bool), brk])
    is_last = jnp.concatenate([brk, jnp.ones((1,), bool)])
    seg_start = lax.cummax(jnp.where(is_start, ar, 0))
    seg_last = jnp.flip(lax.cummin(jnp.flip(jnp.where(is_last, ar, e - 1))))
    t_j = ar - seg_start
    negdeg_e = seg_start - seg_last - 1
    if e < (1 << 18) and n <= (1 << 14):
        key2 = (((jnp.uint32(e) + negdeg_e.astype(jnp.uint32))
                 << jnp.uint32(14)) | src_s.astype(jnp.uint32))
        _, _, dst_packed = lax.sort(
            (t_j, key2, dst_s), num_keys=2, is_stable=False)
    else:
        _, _, _, dst_packed = lax.sort(
            (t_j, negdeg_e, src_s, dst_s), num_keys=3, is_stable=False)
    gather_idx = jnp.concatenate(
        [dst_packed, jnp.zeros((e_pad - e,), jnp.int32)])

    deg = jnp.bincount(src, length=n).astype(jnp.int32)
    if e < (1 << 18) and n <= (1 << 14):
        keyn = (((jnp.uint32(e) - deg.astype(jnp.uint32)) << jnp.uint32(14))
                | jnp.arange(n, dtype=jnp.uint32))
        keyn_s = lax.sort(keyn, is_stable=False)
        node_order = (keyn_s & jnp.uint32((1 << 14) - 1)).astype(jnp.int32)
        deg_desc = (jnp.uint32(e) - (keyn_s >> jnp.uint32(14))).astype(jnp.int32)
    else:
        negdeg_n, node_order = lax.sort(
            (-deg, jnp.arange(n, dtype=jnp.int32)), num_keys=1, is_stable=True)
        deg_desc = -negdeg_n
    deg_pad = jnp.zeros((n_pad,), jnp.int32).at[:n].set(deg_desc)

    x_tm = _sc_gather(input_matrix, gather_idx)

    wih_t = Wih.T
    whh_t = Whh.T
    bias = (bih + bhh).reshape(1, -1)
    h_fin = _lstm_packed(deg_pad, x_tm, wih_t, whh_t, bias, n_pad)

    scat = jnp.concatenate(
        [node_order, jnp.arange(n, n_pad, dtype=jnp.int32)]
    ).reshape(nw, -1, _LANES)
    agg = _sc_scatter(h_fin, scat)

    blk = next((b for b in range(min(512, n), 7, -8)
                if b % 8 == 0 and n % b == 0), None)
    if blk is not None:
        return _out_matmul(input_matrix, agg, W, blk)
    x_pad = jnp.zeros((n_pad, d), input_matrix.dtype).at[:n].set(input_matrix)
    return _out_matmul(x_pad, agg, W, _LANES)[:n]
